# baseline (device time: 19643 ns/iter reference)
import jax
import jax.numpy as jnp
from jax import lax
from jax.experimental import pallas as pl
from jax.experimental.pallas import tpu as pltpu

X_SIZE = 2
K = 4


def kernel(x):
    m, n = x.shape
    half = n // X_SIZE
    rows = m // K

    def body(x_ref, out_ref, sq, rq, sscale, rscale,
             dsend_sems, drecv_sems, ssend_sems, srecv_sems):
        my_x = lax.axis_index("x")
        my_y = lax.axis_index("y")
        my_z = lax.axis_index("z")
        ox = 1 - my_x
        partner = (ox, my_y, my_z)

        barrier_sem = pltpu.get_barrier_semaphore()
        pl.semaphore_signal(
            barrier_sem, inc=1,
            device_id=partner, device_id_type=pl.DeviceIdType.MESH,
        )
        pl.semaphore_wait(barrier_sem, 1)

        rdmas = []
        for h in range(K):
            sl = pl.ds(h * rows, rows)
            blk = x_ref[sl, pl.ds(ox * half, half)]
            amax = jnp.max(jnp.abs(blk), axis=1, keepdims=True)
            inv = 127.0 / jnp.maximum(amax, 1e-30)
            sscale[sl, :] = amax * (1.0 / 127.0)
            sq[sl, :] = jnp.round(blk * inv).astype(jnp.int8)
            rd = pltpu.make_async_remote_copy(
                src_ref=sq.at[sl, :],
                dst_ref=rq.at[sl, :],
                send_sem=dsend_sems.at[h],
                recv_sem=drecv_sems.at[h],
                device_id=partner,
                device_id_type=pl.DeviceIdType.MESH,
            )
            rs = pltpu.make_async_remote_copy(
                src_ref=sscale.at[sl, :],
                dst_ref=rscale.at[sl, :],
                send_sem=ssend_sems.at[h],
                recv_sem=srecv_sems.at[h],
                device_id=partner,
                device_id_type=pl.DeviceIdType.MESH,
            )
            rd.start()
            rs.start()
            rdmas.append((rd, rs))

        out_ref[pl.ds(my_x * m, m), :] = x_ref[:, pl.ds(my_x * half, half)]

        for h, (rd, rs) in enumerate(rdmas):
            rd.wait()
            rs.wait()
            sl = pl.ds(h * rows, rows)
            out_ref[pl.ds(ox * m + h * rows, rows), :] = (
                rq[sl, :].astype(jnp.float32) * rscale[sl, :]
            )

    return pl.pallas_call(
        body,
        out_shape=jax.ShapeDtypeStruct((X_SIZE * m, half), x.dtype),
        in_specs=[pl.BlockSpec(memory_space=pltpu.VMEM)],
        out_specs=pl.BlockSpec(memory_space=pltpu.VMEM),
        scratch_shapes=[
            pltpu.VMEM((m, half), jnp.int8),
            pltpu.VMEM((m, half), jnp.int8),
            pltpu.VMEM((m, 1), jnp.float32),
            pltpu.VMEM((m, 1), jnp.float32),
            pltpu.SemaphoreType.DMA((K,)),
            pltpu.SemaphoreType.DMA((K,)),
            pltpu.SemaphoreType.DMA((K,)),
            pltpu.SemaphoreType.DMA((K,)),
        ],
        compiler_params=pltpu.CompilerParams(collective_id=0),
    )(x)
